# speedup vs baseline: 1.1711x; 1.1711x over previous
import functools

import jax
import jax.numpy as jnp
from jax.experimental import pallas as pl
from jax.experimental.pallas import tpu as pltpu


def _sepconv_kernel(x_ref, wd_ref, wp_ref, m_ref, o_ref, *, H, W, KH, KW,
                    dilation, padding):
    HW = H * W
    xb = x_ref[...].astype(jnp.bfloat16)
    wd = wd_ref[...]
    m = m_ref[...]

    acc = None
    for kh in range(KH):
        dh = kh * dilation - padding
        for kw in range(KW):
            dw = kw * dilation - padding
            t = kh * KW + kw
            shift = dh * W + dw
            if shift == 0:
                patch = xb
            else:
                patch = pltpu.roll(xb, shift=(-shift) % HW, axis=1)
            if dh != 0 or dw != 0:
                patch = patch * m[t:t + 1, :]
            term = patch * wd[:, t:t + 1]
            acc = term if acc is None else acc + term

    out = jnp.dot(wp_ref[...], acc, preferred_element_type=jnp.float32)
    o_ref[...] = out.astype(o_ref.dtype)


def _tap_masks(H, W, KH, KW, dilation, padding):
    lane = jnp.arange(H * W, dtype=jnp.int32)
    hh = lane // W
    ww = lane - hh * W
    rows = []
    for kh in range(KH):
        dh = kh * dilation - padding
        for kw in range(KW):
            dw = kw * dilation - padding
            ok = ((hh + dh >= 0) & (hh + dh < H) &
                  (ww + dw >= 0) & (ww + dw < W))
            rows.append(ok)
    return jnp.stack(rows).astype(jnp.bfloat16)


def kernel(x_nchw, w_dw, w_pw):
    N, C, H, W = x_nchw.shape
    KH, KW, _ = w_dw.shape
    O = w_pw.shape[1]
    HW = H * W
    dilation, padding = 1, 1

    x_flat = x_nchw.reshape(N * C, HW)
    wd = jnp.transpose(w_dw.reshape(KH * KW, C)).astype(jnp.bfloat16)
    wp = jnp.transpose(w_pw).astype(jnp.bfloat16)
    masks = _tap_masks(H, W, KH, KW, dilation, padding)

    kernel_fn = functools.partial(_sepconv_kernel, H=H, W=W, KH=KH, KW=KW,
                                  dilation=dilation, padding=padding)

    out_flat = pl.pallas_call(
        kernel_fn,
        out_shape=jax.ShapeDtypeStruct((N * O, HW), x_nchw.dtype),
        grid_spec=pltpu.PrefetchScalarGridSpec(
            num_scalar_prefetch=0,
            grid=(N,),
            in_specs=[
                pl.BlockSpec((C, HW), lambda g: (g, 0)),
                pl.BlockSpec((C, KH * KW), lambda g: (0, 0)),
                pl.BlockSpec((O, C), lambda g: (0, 0)),
                pl.BlockSpec((KH * KW, HW), lambda g: (0, 0)),
            ],
            out_specs=pl.BlockSpec((O, HW), lambda g: (g, 0)),
        ),
        compiler_params=pltpu.CompilerParams(
            dimension_semantics=("parallel",),
            vmem_limit_bytes=32 << 20),
    )(x_flat, wd, wp, masks)

    return out_flat.reshape(N, O, H, W)
